# baseline (speedup 1.0000x reference)
import jax
import jax.numpy as jnp
from jax.experimental import pallas as pl
from jax.experimental.pallas import tpu as pltpu

_K = 8192
_D = 32
_BN = 128
_CH = 256
_NCH = _K // _CH


def _vq_block(x_ref, x2_ref, w2x_ref, w2_ref, lane_ref, enc_ref, q_ref,
              dist_ref):
    x_blk = x_ref[...]
    x2 = x2_ref[...]

    def dist_chunk_body(k, acc):
        base = k * _CH
        w2x_c = w2x_ref[pl.ds(base, _CH), :]
        s2 = jax.lax.dot_general(
            x_blk, w2x_c, (((1,), (1,)), ((), ())),
            preferred_element_type=jnp.float32)
        d2 = x2 - s2 + w2_ref[:, pl.ds(base, _CH)]
        dist = jnp.sqrt(jnp.maximum(d2, 0.0))
        dist_ref[:, pl.ds(base, _CH)] = dist
        return jnp.minimum(acc, dist)

    acc = jax.lax.fori_loop(
        0, _NCH, dist_chunk_body,
        jnp.full((_BN, _CH), jnp.inf, jnp.float32), unroll=1)
    m = jnp.min(acc, axis=1, keepdims=True)

    def z_chunk_body(k, zacc):
        base = k * _CH
        dist = dist_ref[:, pl.ds(base, _CH)]
        lanes = lane_ref[:, pl.ds(base, _CH)]
        return jnp.minimum(zacc, jnp.where(dist == m, lanes, jnp.float32(_K)))

    zacc = jax.lax.fori_loop(
        0, _NCH, z_chunk_body,
        jnp.full((_BN, _CH), jnp.float32(_K), jnp.float32), unroll=1)
    idx = jnp.min(zacc, axis=1, keepdims=True)

    def enc_chunk_body(k, carry):
        base = k * _CH
        lanes = lane_ref[:, pl.ds(base, _CH)]
        enc_ref[:, pl.ds(base, _CH)] = jnp.where(
            lanes == idx, jnp.float32(1.0), jnp.float32(0.0))
        return carry

    jax.lax.fori_loop(0, _NCH, enc_chunk_body, 0, unroll=1)
    q_ref[...] = 0.5 * jnp.dot(enc_ref[...], w2x_ref[...],
                               preferred_element_type=jnp.float32)


def kernel(x, weight):
    b, c, h, w_sp = x.shape
    x_flat = jnp.transpose(x, (0, 2, 3, 1)).reshape(-1, _D)
    n = x_flat.shape[0]
    x2 = jnp.sum(x_flat * x_flat, axis=1, keepdims=True)
    w2 = jnp.sum(weight * weight, axis=1)[None, :]
    w2x = weight + weight
    lane_row = jnp.arange(_K, dtype=jnp.float32)[None, :]
    grid = n // _BN
    enc, q = pl.pallas_call(
        _vq_block,
        grid=(grid,),
        in_specs=[
            pl.BlockSpec((_BN, _D), lambda i: (i, 0)),
            pl.BlockSpec((_BN, 1), lambda i: (i, 0)),
            pl.BlockSpec((_K, _D), lambda i: (0, 0)),
            pl.BlockSpec((1, _K), lambda i: (0, 0)),
            pl.BlockSpec((1, _K), lambda i: (0, 0)),
        ],
        out_specs=[
            pl.BlockSpec((_BN, _K), lambda i: (i, 0)),
            pl.BlockSpec((_BN, _D), lambda i: (i, 0)),
        ],
        out_shape=[
            jax.ShapeDtypeStruct((n, _K), jnp.float32),
            jax.ShapeDtypeStruct((n, _D), jnp.float32),
        ],
        scratch_shapes=[pltpu.VMEM((_BN, _K), jnp.float32)],
    )(x_flat, x2, w2x, w2, lane_row)
    quantized = jnp.transpose(q.reshape(b, h, w_sp, c), (0, 3, 1, 2))
    return enc, quantized


# speedup vs baseline: 1.8263x; 1.8263x over previous
import jax
import jax.numpy as jnp
from jax.experimental import pallas as pl
from jax.experimental.pallas import tpu as pltpu

_K = 8192
_D = 32
_BN = 128
_CH = 256
_NCH = _K // _CH


def _vq_block(x_ref, x2_ref, w2x_ref, w2_ref, lane_ref, enc_ref, q_ref,
              dist_ref):
    x_blk = x_ref[...]
    x2 = x2_ref[...]

    def dist_chunk_body(k, acc):
        base = k * _CH
        w2x_c = w2x_ref[pl.ds(base, _CH), :]
        s2 = jax.lax.dot_general(
            x_blk, w2x_c, (((1,), (1,)), ((), ())),
            preferred_element_type=jnp.float32)
        d2 = x2 - s2 + w2_ref[:, pl.ds(base, _CH)]
        dist = jnp.sqrt(jnp.maximum(d2, 0.0))
        dist_ref[:, pl.ds(base, _CH)] = dist
        return jnp.minimum(acc, dist)

    acc = jax.lax.fori_loop(
        0, _NCH, dist_chunk_body,
        jnp.full((_BN, _CH), jnp.inf, jnp.float32), unroll=8)
    m = jnp.min(acc, axis=1, keepdims=True)

    def z_chunk_body(k, zacc):
        base = k * _CH
        dist = dist_ref[:, pl.ds(base, _CH)]
        lanes = lane_ref[:, pl.ds(base, _CH)]
        return jnp.minimum(zacc, jnp.where(dist == m, lanes, jnp.float32(_K)))

    zacc = jax.lax.fori_loop(
        0, _NCH, z_chunk_body,
        jnp.full((_BN, _CH), jnp.float32(_K), jnp.float32), unroll=8)
    idx = jnp.min(zacc, axis=1, keepdims=True)

    def enc_chunk_body(k, carry):
        base = k * _CH
        lanes = lane_ref[:, pl.ds(base, _CH)]
        enc_ref[:, pl.ds(base, _CH)] = jnp.where(
            lanes == idx, jnp.float32(1.0), jnp.float32(0.0))
        return carry

    jax.lax.fori_loop(0, _NCH, enc_chunk_body, 0, unroll=8)
    q_ref[...] = 0.5 * jnp.dot(enc_ref[...], w2x_ref[...],
                               preferred_element_type=jnp.float32)


def kernel(x, weight):
    b, c, h, w_sp = x.shape
    x_flat = jnp.transpose(x, (0, 2, 3, 1)).reshape(-1, _D)
    n = x_flat.shape[0]
    x2 = jnp.sum(x_flat * x_flat, axis=1, keepdims=True)
    w2 = jnp.sum(weight * weight, axis=1)[None, :]
    w2x = weight + weight
    lane_row = jnp.arange(_K, dtype=jnp.float32)[None, :]
    grid = n // _BN
    enc, q = pl.pallas_call(
        _vq_block,
        grid=(grid,),
        in_specs=[
            pl.BlockSpec((_BN, _D), lambda i: (i, 0)),
            pl.BlockSpec((_BN, 1), lambda i: (i, 0)),
            pl.BlockSpec((_K, _D), lambda i: (0, 0)),
            pl.BlockSpec((1, _K), lambda i: (0, 0)),
            pl.BlockSpec((1, _K), lambda i: (0, 0)),
        ],
        out_specs=[
            pl.BlockSpec((_BN, _K), lambda i: (i, 0)),
            pl.BlockSpec((_BN, _D), lambda i: (i, 0)),
        ],
        out_shape=[
            jax.ShapeDtypeStruct((n, _K), jnp.float32),
            jax.ShapeDtypeStruct((n, _D), jnp.float32),
        ],
        scratch_shapes=[pltpu.VMEM((_BN, _K), jnp.float32)],
    )(x_flat, x2, w2x, w2, lane_row)
    quantized = jnp.transpose(q.reshape(b, h, w_sp, c), (0, 3, 1, 2))
    return enc, quantized
